# baseline (speedup 1.0000x reference)
import functools

import jax
import jax.numpy as jnp
from jax import lax
from jax.experimental import pallas as pl
from jax.experimental.pallas import tpu as pltpu
from jax.experimental.pallas import tpu_sc as plsc

E = 8
D_IN = 2048
Z_DIM = 2048
B = 128

ROWS = E * Z_DIM
ROW_BLK = 1024


def _matvec_body(w_ref, z_ref, b_ref, y_ref):
    y_ref[...] = (
        jnp.dot(w_ref[...], z_ref[...], preferred_element_type=jnp.float32)
        + b_ref[...]
    )


_matvec = pl.pallas_call(
    _matvec_body,
    grid=(ROWS // ROW_BLK,),
    in_specs=[
        pl.BlockSpec((ROW_BLK, D_IN), lambda i: (i, 0)),
        pl.BlockSpec((D_IN, 1), lambda i: (0, 0)),
        pl.BlockSpec((ROW_BLK, 1), lambda i: (i, 0)),
    ],
    out_specs=pl.BlockSpec((ROW_BLK, 1), lambda i: (i, 0)),
    out_shape=jax.ShapeDtypeStruct((ROWS, 1), jnp.float32),
)


_NW_USED = 16
_B_PER_W = B // _NW_USED

_sc_mesh = plsc.VectorSubcoreMesh(core_axis_name="c", subcore_axis_name="s")


@functools.partial(
    pl.kernel,
    out_type=jax.ShapeDtypeStruct((B, Z_DIM), jnp.float32),
    mesh=_sc_mesh,
    scratch_types=[
        pltpu.VMEM((_B_PER_W,), jnp.int32),
        pltpu.VMEM((_B_PER_W, Z_DIM), jnp.float32),
        pltpu.SemaphoreType.DMA,
    ],
)
def _sc_gather(y_hbm, idx_hbm, out_hbm, idx_v, rows_v, sem):
    wid = lax.axis_index("s") * 2 + lax.axis_index("c")

    @pl.when(wid < _NW_USED)
    def _():
        base = wid * _B_PER_W
        pltpu.sync_copy(idx_hbm.at[pl.ds(base, _B_PER_W)], idx_v)
        pltpu.async_copy(y_hbm.at[idx_v], rows_v, sem).wait()
        pltpu.sync_copy(rows_v, out_hbm.at[pl.ds(base, _B_PER_W)])


_K_SC = 4
_SC_ROWS = _K_SC * Z_DIM
_TC_ROWS = ROWS - _SC_ROWS
_RPW = _SC_ROWS // 32
_NCH = _RPW // 16


@functools.partial(
    pl.kernel,
    out_type=jax.ShapeDtypeStruct((_SC_ROWS * 16,), jnp.float32),
    mesh=_sc_mesh,
    scratch_types=[
        pltpu.VMEM((D_IN,), jnp.float32),
        pltpu.VMEM((2, 16, D_IN), jnp.float32),
        pltpu.VMEM((_RPW * 16,), jnp.float32),
        pltpu.SemaphoreType.DMA,
        pltpu.SemaphoreType.DMA,
    ],
)
def _sc_matvec(w_hbm, z_hbm, yp_hbm, z_v, wbuf, res_v, sem0, sem1):
    wid = lax.axis_index("s") * 2 + lax.axis_index("c")
    row0 = _TC_ROWS + wid * _RPW
    pltpu.sync_copy(z_hbm, z_v)
    sems = (sem0, sem1)
    cps = [None, None]
    cps[0] = pltpu.async_copy(w_hbm.at[pl.ds(row0, 16)], wbuf.at[0], sems[0])
    for c in range(_NCH):
        cur = c % 2
        nxt = (c + 1) % 2
        if c + 1 < _NCH:
            cps[nxt] = pltpu.async_copy(
                w_hbm.at[pl.ds(row0 + (c + 1) * 16, 16)], wbuf.at[nxt],
                sems[nxt])
        cps[cur].wait()

        def body(i, accs, _b=cur):
            z16 = z_v[pl.ds(i * 16, 16)]
            return tuple(
                accs[j] + wbuf[_b, j, pl.ds(i * 16, 16)] * z16
                for j in range(16))
        accs = lax.fori_loop(
            0, D_IN // 16, body,
            tuple(jnp.zeros((16,), jnp.float32) for _ in range(16)))
        for j in range(16):
            res_v[pl.ds((c * 16 + j) * 16, 16)] = accs[j]
    pltpu.sync_copy(res_v, yp_hbm.at[pl.ds(wid * _RPW * 16, _RPW * 16)])


def _reduce_body(p_ref, b_ref, y_ref):
    y_ref[...] = jnp.sum(p_ref[...], axis=1, keepdims=True) + b_ref[...]


_partial_reduce = pl.pallas_call(
    _reduce_body,
    out_shape=jax.ShapeDtypeStruct((_SC_ROWS, 1), jnp.float32),
)


_matvec_tc = pl.pallas_call(
    _matvec_body,
    grid=(_TC_ROWS // ROW_BLK,),
    in_specs=[
        pl.BlockSpec((ROW_BLK, D_IN), lambda i: (i, 0)),
        pl.BlockSpec((D_IN, 1), lambda i: (0, 0)),
        pl.BlockSpec((ROW_BLK, 1), lambda i: (i, 0)),
    ],
    out_specs=pl.BlockSpec((ROW_BLK, 1), lambda i: (i, 0)),
    out_shape=jax.ShapeDtypeStruct((_TC_ROWS, 1), jnp.float32),
)


def kernel(indices, z, W, b):
    idx = indices.astype(jnp.int32)
    w_flat = W.reshape(ROWS, D_IN)
    b_flat = b.reshape(ROWS, 1)
    z_col = z.reshape(D_IN, 1)
    y_part = _sc_matvec(w_flat, z).reshape(_SC_ROWS, 16)
    y_tc = _matvec_tc(w_flat, z_col, b_flat)
    y_sc = _partial_reduce(y_part, b_flat[_TC_ROWS:])
    y = jnp.concatenate([y_tc[:, 0], y_sc[:, 0]]).reshape(E, Z_DIM)
    return _sc_gather(y, idx)


# speedup vs baseline: 1.1342x; 1.1342x over previous
import functools

import jax
import jax.numpy as jnp
from jax import lax
from jax.experimental import pallas as pl
from jax.experimental.pallas import tpu as pltpu
from jax.experimental.pallas import tpu_sc as plsc

E = 8
D_IN = 2048
Z_DIM = 2048
B = 128

ROWS = E * Z_DIM
ROW_BLK = 1024


def _matvec_body(w_ref, z_ref, b_ref, y_ref):
    y_ref[...] = (
        jnp.dot(w_ref[...], z_ref[...], preferred_element_type=jnp.float32)
        + b_ref[...]
    )


_matvec = pl.pallas_call(
    _matvec_body,
    grid=(ROWS // ROW_BLK,),
    in_specs=[
        pl.BlockSpec((ROW_BLK, D_IN), lambda i: (i, 0)),
        pl.BlockSpec((D_IN, 1), lambda i: (0, 0)),
        pl.BlockSpec((ROW_BLK, 1), lambda i: (i, 0)),
    ],
    out_specs=pl.BlockSpec((ROW_BLK, 1), lambda i: (i, 0)),
    out_shape=jax.ShapeDtypeStruct((ROWS, 1), jnp.float32),
)


_NW_USED = 16
_B_PER_W = B // _NW_USED

_sc_mesh = plsc.VectorSubcoreMesh(core_axis_name="c", subcore_axis_name="s")


@functools.partial(
    pl.kernel,
    out_type=jax.ShapeDtypeStruct((B, Z_DIM), jnp.float32),
    mesh=_sc_mesh,
    scratch_types=[
        pltpu.VMEM((_B_PER_W,), jnp.int32),
        pltpu.VMEM((_B_PER_W, Z_DIM), jnp.float32),
        pltpu.SemaphoreType.DMA,
    ],
)
def _sc_gather(y_hbm, idx_hbm, out_hbm, idx_v, rows_v, sem):
    wid = lax.axis_index("s") * 2 + lax.axis_index("c")

    @pl.when(wid < _NW_USED)
    def _():
        base = wid * _B_PER_W
        pltpu.sync_copy(idx_hbm.at[pl.ds(base, _B_PER_W)], idx_v)
        pltpu.async_copy(y_hbm.at[idx_v], rows_v, sem).wait()
        pltpu.sync_copy(rows_v, out_hbm.at[pl.ds(base, _B_PER_W)])


_HALF = ROWS // 2


def _matvec2_body(w1_ref, w2_ref, z_ref, b1_ref, b2_ref, y1_ref, y2_ref):
    zv = z_ref[...]
    y1_ref[...] = (
        jnp.dot(w1_ref[...], zv, preferred_element_type=jnp.float32)
        + b1_ref[...])
    y2_ref[...] = (
        jnp.dot(w2_ref[...], zv, preferred_element_type=jnp.float32)
        + b2_ref[...])


_matvec2 = pl.pallas_call(
    _matvec2_body,
    grid=(_HALF // ROW_BLK,),
    in_specs=[
        pl.BlockSpec((ROW_BLK, D_IN), lambda i: (i, 0)),
        pl.BlockSpec((ROW_BLK, D_IN), lambda i: (i + _HALF // ROW_BLK, 0)),
        pl.BlockSpec((D_IN, 1), lambda i: (0, 0)),
        pl.BlockSpec((ROW_BLK, 1), lambda i: (i, 0)),
        pl.BlockSpec((ROW_BLK, 1), lambda i: (i + _HALF // ROW_BLK, 0)),
    ],
    out_specs=[
        pl.BlockSpec((ROW_BLK, 1), lambda i: (i, 0)),
        pl.BlockSpec((ROW_BLK, 1), lambda i: (i, 0)),
    ],
    out_shape=[
        jax.ShapeDtypeStruct((_HALF, 1), jnp.float32),
        jax.ShapeDtypeStruct((_HALF, 1), jnp.float32),
    ],
)


def kernel(indices, z, W, b):
    idx = indices.astype(jnp.int32)
    w_flat = W.reshape(ROWS, D_IN)
    b_flat = b.reshape(ROWS, 1)
    z_col = z.reshape(D_IN, 1)
    y1, y2 = _matvec2(w_flat, w_flat, z_col, b_flat, b_flat)
    y = jnp.concatenate([y1[:, 0], y2[:, 0]]).reshape(E, Z_DIM)
    return _sc_gather(y, idx)


# speedup vs baseline: 1.1669x; 1.0289x over previous
import functools

import jax
import jax.numpy as jnp
from jax import lax
from jax.experimental import pallas as pl
from jax.experimental.pallas import tpu as pltpu
from jax.experimental.pallas import tpu_sc as plsc

E = 8
D_IN = 2048
Z_DIM = 2048
B = 128

ROWS = E * Z_DIM
ROW_BLK = 1024


def _matvec_body(w_ref, z_ref, b_ref, y_ref):
    y_ref[...] = (
        jnp.dot(w_ref[...], z_ref[...], preferred_element_type=jnp.float32)
        + b_ref[...]
    )


_matvec = pl.pallas_call(
    _matvec_body,
    grid=(ROWS // ROW_BLK,),
    in_specs=[
        pl.BlockSpec((ROW_BLK, D_IN), lambda i: (i, 0)),
        pl.BlockSpec((D_IN, 1), lambda i: (0, 0)),
        pl.BlockSpec((ROW_BLK, 1), lambda i: (i, 0)),
    ],
    out_specs=pl.BlockSpec((ROW_BLK, 1), lambda i: (i, 0)),
    out_shape=jax.ShapeDtypeStruct((ROWS, 1), jnp.float32),
)


_B_PER_W = 4

_sc_mesh = plsc.VectorSubcoreMesh(core_axis_name="c", subcore_axis_name="s")


@functools.partial(
    pl.kernel,
    out_type=jax.ShapeDtypeStruct((B, Z_DIM), jnp.float32),
    mesh=_sc_mesh,
    scratch_types=[
        pltpu.VMEM((_B_PER_W,), jnp.int32),
        pltpu.VMEM((_B_PER_W, Z_DIM), jnp.float32),
        pltpu.SemaphoreType.DMA,
    ],
)
def _sc_gather(y_hbm, idxpad_hbm, out_hbm, idx_v, rows_v, sem):
    wid = lax.axis_index("s") * 2 + lax.axis_index("c")
    pltpu.sync_copy(idxpad_hbm.at[pl.ds(wid * 8, _B_PER_W)], idx_v)
    pltpu.async_copy(y_hbm.at[idx_v], rows_v, sem).wait()
    pltpu.sync_copy(rows_v, out_hbm.at[pl.ds(wid * _B_PER_W, _B_PER_W)])


_HALF = ROWS // 2


def _matvec2_body(w1_ref, w2_ref, z_ref, b1_ref, b2_ref, y1_ref, y2_ref):
    zv = z_ref[...]
    y1_ref[...] = (
        jnp.dot(w1_ref[...], zv, preferred_element_type=jnp.float32)
        + b1_ref[...])
    y2_ref[...] = (
        jnp.dot(w2_ref[...], zv, preferred_element_type=jnp.float32)
        + b2_ref[...])


_matvec2 = pl.pallas_call(
    _matvec2_body,
    grid=(_HALF // ROW_BLK,),
    in_specs=[
        pl.BlockSpec((ROW_BLK, D_IN), lambda i: (i, 0)),
        pl.BlockSpec((ROW_BLK, D_IN), lambda i: (i + _HALF // ROW_BLK, 0)),
        pl.BlockSpec((D_IN, 1), lambda i: (0, 0)),
        pl.BlockSpec((ROW_BLK, 1), lambda i: (i, 0)),
        pl.BlockSpec((ROW_BLK, 1), lambda i: (i + _HALF // ROW_BLK, 0)),
    ],
    out_specs=[
        pl.BlockSpec((ROW_BLK, 1), lambda i: (i, 0)),
        pl.BlockSpec((ROW_BLK, 1), lambda i: (i, 0)),
    ],
    out_shape=[
        jax.ShapeDtypeStruct((_HALF, 1), jnp.float32),
        jax.ShapeDtypeStruct((_HALF, 1), jnp.float32),
    ],
)


def kernel(indices, z, W, b):
    idx = indices.astype(jnp.int32)
    w_flat = W.reshape(ROWS, D_IN)
    b_flat = b.reshape(ROWS, 1)
    z_col = z.reshape(D_IN, 1)
    idx_pad = jnp.pad(idx.reshape(32, _B_PER_W),
                      ((0, 0), (0, 8 - _B_PER_W))).reshape(-1)
    y = _matvec(w_flat, z_col, b_flat).reshape(E, Z_DIM)
    return _sc_gather(y, idx_pad)


# speedup vs baseline: 1.1674x; 1.0005x over previous
import functools

import jax
import jax.numpy as jnp
from jax import lax
from jax.experimental import pallas as pl
from jax.experimental.pallas import tpu as pltpu
from jax.experimental.pallas import tpu_sc as plsc

E = 8
D_IN = 2048
Z_DIM = 2048
B = 128

ROWS = E * Z_DIM
ROW_BLK = 1024


def _matvec_body(w_ref, z_ref, b_ref, y_ref):
    y_ref[...] = (
        jnp.dot(w_ref[...], z_ref[...], preferred_element_type=jnp.float32)
        + b_ref[...]
    )


_matvec = pl.pallas_call(
    _matvec_body,
    grid=(ROWS // ROW_BLK,),
    in_specs=[
        pl.BlockSpec((ROW_BLK, D_IN), lambda i: (i, 0)),
        pl.BlockSpec((D_IN, 1), lambda i: (0, 0)),
        pl.BlockSpec((ROW_BLK, 1), lambda i: (i, 0)),
    ],
    out_specs=pl.BlockSpec((ROW_BLK, 1), lambda i: (i, 0)),
    out_shape=jax.ShapeDtypeStruct((ROWS, 1), jnp.float32),
)


_B_PER_W = 4

_sc_mesh = plsc.VectorSubcoreMesh(core_axis_name="c", subcore_axis_name="s")


@functools.partial(
    pl.kernel,
    out_type=jax.ShapeDtypeStruct((B, Z_DIM), jnp.float32),
    mesh=_sc_mesh,
    scratch_types=[
        pltpu.VMEM((_B_PER_W,), jnp.int32),
        pltpu.VMEM((_B_PER_W, Z_DIM), jnp.float32),
        pltpu.SemaphoreType.DMA,
    ],
)
def _sc_gather(y_hbm, idxpad_hbm, out_hbm, idx_v, rows_v, sem):
    wid = lax.axis_index("s") * 2 + lax.axis_index("c")
    pltpu.sync_copy(idxpad_hbm.at[pl.ds(wid * 8, _B_PER_W)], idx_v)
    pltpu.async_copy(y_hbm.at[idx_v], rows_v, sem).wait()
    pltpu.sync_copy(rows_v, out_hbm.at[pl.ds(wid * _B_PER_W, _B_PER_W)])


def kernel(indices, z, W, b):
    idx = indices.astype(jnp.int32)
    w_flat = W.reshape(ROWS, D_IN)
    b_flat = b.reshape(ROWS, 1)
    z_col = z.reshape(D_IN, 1)
    idx_pad = jnp.pad(idx.reshape(32, _B_PER_W),
                      ((0, 0), (0, 8 - _B_PER_W))).reshape(-1)
    y = _matvec(w_flat, z_col, b_flat).reshape(E, Z_DIM)
    return _sc_gather(y, idx_pad)
